# Initial kernel scaffold; baseline (speedup 1.0000x reference)
import jax
import jax.numpy as jnp
from jax.experimental import pallas as pl


def kernel(x, Wih0, b0, Whh0, Wih1, b1, Whh1):
    raise NotImplementedError("write your pallas kernel here")


# speedup vs baseline: 4.7738x; 4.7738x over previous
import functools

import jax
import jax.numpy as jnp
from jax.experimental import pallas as pl
from jax.experimental.pallas import tpu as pltpu

B, S, D, H, L = 64, 512, 1024, 1024, 2

T_STEPS = 8
M_BLK = 1024
N_CORES = 2


def _pre0_body(x_ref, w_ref, b_ref, o_ref):
    o_ref[...] = (
        jnp.dot(x_ref[...], w_ref[...], preferred_element_type=jnp.float32)
        + b_ref[...]
    )


def _rnn_body(pre0_ref, whh0_ref, wih1_ref, b1_ref, whh1_ref,
              out_ref, hlast_ref, h0_s, h1_s, *, n_t):
    t = pl.program_id(1)

    @pl.when(t == 0)
    def _():
        h0_s[...] = jnp.zeros_like(h0_s)
        h1_s[...] = jnp.zeros_like(h1_s)

    h0 = h0_s[...]
    h1 = h1_s[...]
    whh0 = whh0_ref[...]
    wih1 = wih1_ref[...]
    whh1 = whh1_ref[...]
    b1 = b1_ref[...]

    for tt in range(T_STEPS):
        a0 = pre0_ref[:, tt, :] + jnp.dot(
            h0, whh0, preferred_element_type=jnp.float32)
        h0 = jnp.tanh(a0)
        a1 = (
            jnp.dot(h0, wih1, preferred_element_type=jnp.float32)
            + b1
            + jnp.dot(h1, whh1, preferred_element_type=jnp.float32)
        )
        h1 = jnp.tanh(a1)
        out_ref[:, tt, :] = h1

    h0_s[...] = h0
    h1_s[...] = h1

    @pl.when(t == n_t - 1)
    def _():
        hlast_ref[0, :, :] = h0
        hlast_ref[1, :, :] = h1


def kernel(x, Wih0, b0, Whh0, Wih1, b1, Whh1):
    b0_2d = b0.reshape(1, H)
    b1_2d = b1.reshape(1, H)

    xf = x.reshape(B * S, D)
    m_tiles = (B * S) // M_BLK
    m_half = m_tiles // N_CORES
    pre0 = pl.pallas_call(
        _pre0_body,
        grid=(N_CORES, m_half),
        in_specs=[
            pl.BlockSpec((M_BLK, D), lambda i, j: (i * m_half + j, 0)),
            pl.BlockSpec((D, H), lambda i, j: (0, 0)),
            pl.BlockSpec((1, H), lambda i, j: (0, 0)),
        ],
        out_specs=pl.BlockSpec((M_BLK, H), lambda i, j: (i * m_half + j, 0)),
        out_shape=jax.ShapeDtypeStruct((B * S, H), jnp.float32),
        compiler_params=pltpu.CompilerParams(
            dimension_semantics=("parallel", "arbitrary"),
        ),
        name="rnn_pre0",
    )(xf, Wih0, b0_2d).reshape(B, S, H)

    n_t = S // T_STEPS
    b_half = B // N_CORES
    out, hlast = pl.pallas_call(
        functools.partial(_rnn_body, n_t=n_t),
        grid=(N_CORES, n_t),
        in_specs=[
            pl.BlockSpec((b_half, T_STEPS, H), lambda i, t: (i, t, 0)),
            pl.BlockSpec((H, H), lambda i, t: (0, 0)),
            pl.BlockSpec((H, H), lambda i, t: (0, 0)),
            pl.BlockSpec((1, H), lambda i, t: (0, 0)),
            pl.BlockSpec((H, H), lambda i, t: (0, 0)),
        ],
        out_specs=[
            pl.BlockSpec((b_half, T_STEPS, H), lambda i, t: (i, t, 0)),
            pl.BlockSpec((L, b_half, H), lambda i, t: (0, i, 0)),
        ],
        out_shape=[
            jax.ShapeDtypeStruct((B, S, H), jnp.float32),
            jax.ShapeDtypeStruct((L, B, H), jnp.float32),
        ],
        scratch_shapes=[
            pltpu.VMEM((b_half, H), jnp.float32),
            pltpu.VMEM((b_half, H), jnp.float32),
        ],
        compiler_params=pltpu.CompilerParams(
            dimension_semantics=("parallel", "arbitrary"),
        ),
        name="rnn_scan",
    )(pre0, Whh0, Wih1, b1_2d, Whh1)
    return out, hlast


# speedup vs baseline: 9.2362x; 1.9348x over previous
import functools

import jax
import jax.numpy as jnp
from jax.experimental import pallas as pl
from jax.experimental.pallas import tpu as pltpu

B, S, D, H, L = 64, 512, 1024, 1024, 2

T_STEPS = 8
M_BLK = 1024


def _pre0_body(x_ref, w_ref, b_ref, o_ref):
    o_ref[...] = (
        jnp.dot(x_ref[...], w_ref[...], preferred_element_type=jnp.float32)
        + b_ref[...]
    )


def _rnn_body(pre0_ref, whh0_ref, wih1_ref, b1_ref, whh1_ref,
              out_ref, hlast_ref, h0_s, h1_s, h0blk_s, p1_s, *, n_t):
    t = pl.program_id(0)

    @pl.when(t == 0)
    def _():
        h0_s[...] = jnp.zeros_like(h0_s)
        h1_s[...] = jnp.zeros_like(h1_s)

    whh0 = whh0_ref[...]
    whh1 = whh1_ref[...]

    h0 = h0_s[...]
    for tt in range(T_STEPS):
        h0 = jnp.tanh(pre0_ref[:, tt, :] + jnp.dot(
            h0, whh0, preferred_element_type=jnp.float32))
        h0blk_s[tt * B:(tt + 1) * B, :] = h0
    h0_s[...] = h0

    p1_s[...] = (
        jnp.dot(h0blk_s[...], wih1_ref[...],
                preferred_element_type=jnp.float32)
        + b1_ref[...]
    )

    h1 = h1_s[...]
    for tt in range(T_STEPS):
        h1 = jnp.tanh(p1_s[tt * B:(tt + 1) * B, :] + jnp.dot(
            h1, whh1, preferred_element_type=jnp.float32))
        out_ref[:, tt, :] = h1
    h1_s[...] = h1

    @pl.when(t == n_t - 1)
    def _():
        hlast_ref[0, :, :] = h0
        hlast_ref[1, :, :] = h1


def kernel(x, Wih0, b0, Whh0, Wih1, b1, Whh1):
    b0_2d = b0.reshape(1, H)
    b1_2d = b1.reshape(1, H)

    xf = x.reshape(B * S, D)
    m_tiles = (B * S) // M_BLK
    pre0 = pl.pallas_call(
        _pre0_body,
        grid=(m_tiles,),
        in_specs=[
            pl.BlockSpec((M_BLK, D), lambda j: (j, 0)),
            pl.BlockSpec((D, H), lambda j: (0, 0)),
            pl.BlockSpec((1, H), lambda j: (0, 0)),
        ],
        out_specs=pl.BlockSpec((M_BLK, H), lambda j: (j, 0)),
        out_shape=jax.ShapeDtypeStruct((B * S, H), jnp.float32),
        compiler_params=pltpu.CompilerParams(
            dimension_semantics=("arbitrary",),
        ),
        name="rnn_pre0",
    )(xf, Wih0, b0_2d).reshape(B, S, H)

    n_t = S // T_STEPS
    out, hlast = pl.pallas_call(
        functools.partial(_rnn_body, n_t=n_t),
        grid=(n_t,),
        in_specs=[
            pl.BlockSpec((B, T_STEPS, H), lambda t: (0, t, 0)),
            pl.BlockSpec((H, H), lambda t: (0, 0)),
            pl.BlockSpec((H, H), lambda t: (0, 0)),
            pl.BlockSpec((1, H), lambda t: (0, 0)),
            pl.BlockSpec((H, H), lambda t: (0, 0)),
        ],
        out_specs=[
            pl.BlockSpec((B, T_STEPS, H), lambda t: (0, t, 0)),
            pl.BlockSpec((L, B, H), lambda t: (0, 0, 0)),
        ],
        out_shape=[
            jax.ShapeDtypeStruct((B, S, H), jnp.float32),
            jax.ShapeDtypeStruct((L, B, H), jnp.float32),
        ],
        scratch_shapes=[
            pltpu.VMEM((B, H), jnp.float32),
            pltpu.VMEM((B, H), jnp.float32),
            pltpu.VMEM((T_STEPS * B, H), jnp.float32),
            pltpu.VMEM((T_STEPS * B, H), jnp.float32),
        ],
        compiler_params=pltpu.CompilerParams(
            dimension_semantics=("arbitrary",),
        ),
        name="rnn_scan",
    )(pre0, Whh0, Wih1, b1_2d, Whh1)
    return out, hlast
